# Initial kernel scaffold; baseline (speedup 1.0000x reference)
import jax
import jax.numpy as jnp
from jax.experimental import pallas as pl


def kernel(data, emb0, emb1):
    raise NotImplementedError("write your pallas kernel here")


# speedup vs baseline: 1.5983x; 1.5983x over previous
import functools

import jax
import jax.numpy as jnp
from jax import lax
from jax.experimental import pallas as pl
from jax.experimental.pallas import tpu as pltpu
from jax.experimental.pallas import tpu_sc as plsc

_VOCAB = 1000000
_DIM = 64
_NEG = 5
_B = 16384

_NC = 2
_NS = 16
_NW = _NC * _NS
_ROWS_PER_W = _B // _NW
_CH = 128
_N_CHUNKS = _ROWS_PER_W // _CH
_NIDX = 2 + _NEG


def _sc_body(emb0_hbm, emb1_hbm, idx_hbm, out_hbm, idx_v, w_v, c_v, n_v,
             out_v, sem):
    wid = lax.axis_index("s") * _NC + lax.axis_index("c")
    base = wid * _ROWS_PER_W
    lane = lax.iota(jnp.int32, 16)

    def chunk_body(chunk, _):
        cbase = base + chunk * _CH
        pltpu.sync_copy(idx_hbm.at[:, pl.ds(cbase, _CH)], idx_v)
        cps = [
            pltpu.async_copy(emb0_hbm.at[idx_v.at[0]], w_v, sem),
            pltpu.async_copy(emb1_hbm.at[idx_v.at[1]], c_v, sem),
        ]
        for k in range(_NEG):
            cps.append(
                pltpu.async_copy(emb1_hbm.at[idx_v.at[2 + k]], n_v.at[k], sem))
        for cp in cps:
            cp.wait()

        def group_body(g, _):
            rows = lane + g * 16
            accs = [jnp.zeros((16,), jnp.float32) for _ in range(6)]
            for d in range(_DIM):
                dd = jnp.full((16,), d, jnp.int32)
                w = plsc.load_gather(w_v, [rows, dd])
                c = plsc.load_gather(c_v, [rows, dd])
                accs[0] = accs[0] + w * c
                for k in range(_NEG):
                    kk = jnp.full((16,), k, jnp.int32)
                    nk = plsc.load_gather(n_v, [kk, rows, dd])
                    accs[1 + k] = accs[1 + k] + w * nk
            for t in range(6):
                out_v[t, pl.ds(g * 16, 16)] = accs[t]
            return 0

        lax.fori_loop(0, _CH // 16, group_body, 0)
        pltpu.sync_copy(out_v, out_hbm.at[:, pl.ds(cbase, _CH)])
        return 0

    lax.fori_loop(0, _N_CHUNKS, chunk_body, 0)


def _sc_ips(emb0, emb1, idx_all):
    fn = pl.kernel(
        _sc_body,
        out_type=jax.ShapeDtypeStruct((6, _B), jnp.float32),
        mesh=plsc.VectorSubcoreMesh(core_axis_name="c", subcore_axis_name="s"),
        scratch_types=[
            pltpu.VMEM((_NIDX, _CH), jnp.int32),
            pltpu.VMEM((_CH, _DIM), jnp.float32),
            pltpu.VMEM((_CH, _DIM), jnp.float32),
            pltpu.VMEM((_NEG, _CH, _DIM), jnp.float32),
            pltpu.VMEM((6, _CH), jnp.float32),
            pltpu.SemaphoreType.DMA,
        ],
        compiler_params=pltpu.CompilerParams(
            use_tc_tiling_on_sc=False, needs_layout_passes=False),
    )
    return fn(emb0, emb1, idx_all)


def _loss_body(ips_ref, mask_ref, out_ref):
    ips = ips_ref[...]
    m = mask_ref[...]
    pos_y = jnp.clip(ips[0:1, :], -10.0, 10.0)
    neg_y = jnp.clip(-ips[1:6, :], -10.0, 10.0)
    pos_l = jnp.log1p(jnp.exp(-pos_y))
    neg_l = jnp.log1p(jnp.exp(-neg_y)) * m
    out_ref[0, 0] = jnp.sum(pos_l) + jnp.sum(neg_l)


def _tc_loss(ips, mask):
    return pl.pallas_call(
        _loss_body,
        out_shape=jax.ShapeDtypeStruct((1, 1), jnp.float32),
        out_specs=pl.BlockSpec(memory_space=pltpu.SMEM),
    )(ips, mask)


def kernel(data, emb0, emb1):
    cols = data.T
    idx_all = jnp.concatenate([cols[1:2], cols[0:1], cols[2:2 + _NEG]], axis=0)
    mask = cols[2 + _NEG:].astype(jnp.float32)
    ips = _sc_ips(emb0, emb1, idx_all)
    loss = _tc_loss(ips, mask)
    return loss[0, 0]


# speedup vs baseline: 1.5994x; 1.0007x over previous
import jax
import jax.numpy as jnp
from jax import lax
from jax.experimental import pallas as pl
from jax.experimental.pallas import tpu as pltpu
from jax.experimental.pallas import tpu_sc as plsc

_VOCAB = 1000000
_DIM = 64
_NEG = 5
_B = 16384

_NC = 2
_NS = 16
_NW = _NC * _NS
_ROWS_PER_W = _B // _NW
_CH = 128
_N_CHUNKS = _ROWS_PER_W // _CH
_NIDX = 2 + _NEG


def _sc_body(emb0_hbm, emb1_hbm, pidx_hbm, half_hbm, out_hbm,
             pidx_v, half_v, w_v, c_v, n_v, out_v, sem):
    wid = lax.axis_index("s") * _NC + lax.axis_index("c")
    base = wid * _ROWS_PER_W
    lane = lax.iota(jnp.int32, 16)

    def chunk_body(chunk, _):
        cbase = base + chunk * _CH
        pltpu.sync_copy(pidx_hbm.at[:, pl.ds(cbase, _CH)], pidx_v)
        pltpu.sync_copy(half_hbm.at[:, pl.ds(cbase, _CH)], half_v)
        cps = [
            pltpu.async_copy(emb0_hbm.at[pidx_v.at[0]], w_v, sem),
            pltpu.async_copy(emb1_hbm.at[pidx_v.at[1]], c_v, sem),
        ]
        for k in range(_NEG):
            cps.append(
                pltpu.async_copy(emb1_hbm.at[pidx_v.at[2 + k]], n_v.at[k],
                                 sem))
        for cp in cps:
            cp.wait()

        def group_body(g, _):
            rows = lane + g * 16
            off = [half_v[t, pl.ds(g * 16, 16)] * 64 for t in range(_NIDX)]
            accs = [jnp.zeros((16,), jnp.float32) for _ in range(6)]
            for d in range(_DIM):
                w = plsc.load_gather(w_v, [rows, off[0] + d])
                c = plsc.load_gather(c_v, [rows, off[1] + d])
                accs[0] = accs[0] + w * c
                for k in range(_NEG):
                    kk = jnp.full((16,), k, jnp.int32)
                    nk = plsc.load_gather(n_v, [kk, rows, off[2 + k] + d])
                    accs[1 + k] = accs[1 + k] + w * nk
            for t in range(6):
                out_v[t, pl.ds(g * 16, 16)] = accs[t]
            return 0

        lax.fori_loop(0, _CH // 16, group_body, 0)
        pltpu.sync_copy(out_v, out_hbm.at[:, pl.ds(cbase, _CH)])
        return 0

    lax.fori_loop(0, _N_CHUNKS, chunk_body, 0)


def _sc_ips(emb0_2w, emb1_2w, pidx_all, half_all):
    fn = pl.kernel(
        _sc_body,
        out_type=jax.ShapeDtypeStruct((6, _B), jnp.float32),
        mesh=plsc.VectorSubcoreMesh(core_axis_name="c", subcore_axis_name="s"),
        scratch_types=[
            pltpu.VMEM((_NIDX, _CH), jnp.int32),
            pltpu.VMEM((_NIDX, _CH), jnp.int32),
            pltpu.VMEM((_CH, 2 * _DIM), jnp.float32),
            pltpu.VMEM((_CH, 2 * _DIM), jnp.float32),
            pltpu.VMEM((_NEG, _CH, 2 * _DIM), jnp.float32),
            pltpu.VMEM((6, _CH), jnp.float32),
            pltpu.SemaphoreType.DMA,
        ],
        compiler_params=pltpu.CompilerParams(needs_layout_passes=False),
    )
    return fn(emb0_2w, emb1_2w, pidx_all, half_all)


def _loss_body(ips_ref, mask_ref, out_ref):
    ips = ips_ref[...]
    m = mask_ref[...]
    pos_y = jnp.clip(ips[0:1, :], -10.0, 10.0)
    neg_y = jnp.clip(-ips[1:6, :], -10.0, 10.0)
    pos_l = jnp.log1p(jnp.exp(-pos_y))
    neg_l = jnp.log1p(jnp.exp(-neg_y)) * m
    out_ref[0, 0] = jnp.sum(pos_l) + jnp.sum(neg_l)


def _tc_loss(ips, mask):
    return pl.pallas_call(
        _loss_body,
        out_shape=jax.ShapeDtypeStruct((1, 1), jnp.float32),
        out_specs=pl.BlockSpec(memory_space=pltpu.SMEM),
    )(ips, mask)


def kernel(data, emb0, emb1):
    cols = data.T
    idx_all = jnp.concatenate([cols[1:2], cols[0:1], cols[2:2 + _NEG]],
                              axis=0)
    pidx_all = idx_all >> 1
    half_all = idx_all & 1
    mask = cols[2 + _NEG:].astype(jnp.float32)
    emb0_2w = emb0[:_VOCAB].reshape(_VOCAB // 2, 2 * _DIM)
    emb1_2w = emb1.reshape(_VOCAB // 2, 2 * _DIM)
    ips = _sc_ips(emb0_2w, emb1_2w, pidx_all, half_all)
    loss = _tc_loss(ips, mask)
    return loss[0, 0]


# speedup vs baseline: 2.4554x; 1.5352x over previous
import jax
import jax.numpy as jnp
from jax import lax
from jax.experimental import pallas as pl
from jax.experimental.pallas import tpu as pltpu
from jax.experimental.pallas import tpu_sc as plsc

_VOCAB = 1000000
_DIM = 64
_NEG = 5
_B = 16384

_NC = 2
_NS = 16
_NW = _NC * _NS
_ROWS_PER_W = _B // _NW
_CH = 128
_N_CHUNKS = _ROWS_PER_W // _CH
_NIDX = 2 + _NEG


def _sc_body(emb0_hbm, emb1_hbm, pidx_hbm, half_hbm, out_hbm,
             pidx_v, half_v, w_v, c_v, n_v, out_v, sem):
    wid = lax.axis_index("s") * _NC + lax.axis_index("c")
    base = wid * _ROWS_PER_W
    lane = lax.iota(jnp.int32, 16)

    def chunk_body(chunk, _):
        cbase = base + chunk * _CH
        pltpu.sync_copy(pidx_hbm.at[:, pl.ds(cbase, _CH)], pidx_v)
        pltpu.sync_copy(half_hbm.at[:, pl.ds(cbase, _CH)], half_v)
        cps = [
            pltpu.async_copy(emb0_hbm.at[pidx_v.at[0]], w_v, sem),
            pltpu.async_copy(emb1_hbm.at[pidx_v.at[1]], c_v, sem),
        ]
        for k in range(_NEG):
            cps.append(
                pltpu.async_copy(emb1_hbm.at[pidx_v.at[2 + k]], n_v.at[k],
                                 sem))
        for cp in cps:
            cp.wait()

        def group_body(g, _):
            rows = lane + g * 16
            off = [half_v[t, pl.ds(g * 16, 16)] * 64 for t in range(_NIDX)]
            accs = [jnp.zeros((16,), jnp.float32) for _ in range(6)]
            for d in range(_DIM):
                w = plsc.load_gather(w_v, [rows, off[0] + d])
                c = plsc.load_gather(c_v, [rows, off[1] + d])
                accs[0] = accs[0] + w * c
                for k in range(_NEG):
                    kk = jnp.full((16,), k, jnp.int32)
                    nk = plsc.load_gather(n_v, [kk, rows, off[2 + k] + d])
                    accs[1 + k] = accs[1 + k] + w * nk
            for t in range(6):
                out_v[t, pl.ds(g * 16, 16)] = accs[t]
            return 0

        lax.fori_loop(0, _CH // 16, group_body, 0)
        pltpu.sync_copy(out_v, out_hbm.at[:, pl.ds(cbase, _CH)])
        return 0

    lax.fori_loop(0, _N_CHUNKS, chunk_body, 0)


def _sc_ips(emb0_2w, emb1_2w, pidx_all, half_all):
    fn = pl.kernel(
        _sc_body,
        out_type=jax.ShapeDtypeStruct((6, _B), jnp.float32),
        mesh=plsc.VectorSubcoreMesh(core_axis_name="c", subcore_axis_name="s"),
        scratch_types=[
            pltpu.VMEM((_NIDX, _CH), jnp.int32),
            pltpu.VMEM((_NIDX, _CH), jnp.int32),
            pltpu.VMEM((_CH, 2 * _DIM), jnp.float32),
            pltpu.VMEM((_CH, 2 * _DIM), jnp.float32),
            pltpu.VMEM((_NEG, _CH, 2 * _DIM), jnp.float32),
            pltpu.VMEM((6, _CH), jnp.float32),
            pltpu.SemaphoreType.DMA,
        ],
        compiler_params=pltpu.CompilerParams(needs_layout_passes=False),
    )
    return fn(emb0_2w, emb1_2w, pidx_all, half_all)


_TRC = 2048
_NBLK = 245
_HALF = _TRC * _NBLK


def _tr_body(lo_ref, hi_ref, out_ref):
    out_ref[:, 0:_DIM] = lo_ref[...].T
    out_ref[:, _DIM:2 * _DIM] = hi_ref[...].T


def _to_rows(emb_t):
    hi_last = _VOCAB // _TRC
    return pl.pallas_call(
        _tr_body,
        grid=(_NBLK,),
        in_specs=[
            pl.BlockSpec((_DIM, _TRC), lambda i: (0, i)),
            pl.BlockSpec((_DIM, _TRC),
                         lambda i: (0, jnp.minimum(i + _NBLK, hi_last))),
        ],
        out_specs=pl.BlockSpec((_TRC, 128), lambda i: (i, 0)),
        out_shape=jax.ShapeDtypeStruct((_HALF, 128), jnp.float32),
    )(emb_t, emb_t)


def _loss_body(ips_ref, mask_ref, out_ref):
    ips = ips_ref[...]
    m = mask_ref[...]
    pos_y = jnp.clip(ips[0:1, :], -10.0, 10.0)
    neg_y = jnp.clip(-ips[1:6, :], -10.0, 10.0)
    pos_l = jnp.log1p(jnp.exp(-pos_y))
    neg_l = jnp.log1p(jnp.exp(-neg_y)) * m
    out_ref[0, 0] = jnp.sum(pos_l) + jnp.sum(neg_l)


def _tc_loss(ips, mask):
    return pl.pallas_call(
        _loss_body,
        out_shape=jax.ShapeDtypeStruct((1, 1), jnp.float32),
        out_specs=pl.BlockSpec(memory_space=pltpu.SMEM),
    )(ips, mask)


def kernel(data, emb0, emb1):
    cols = data.T
    idx_all = jnp.concatenate([cols[1:2], cols[0:1], cols[2:2 + _NEG]],
                              axis=0)
    half_all = (idx_all >= _HALF).astype(jnp.int32)
    pidx_all = idx_all - half_all * _HALF
    mask = cols[2 + _NEG:].astype(jnp.float32)
    emb0_2w = _to_rows(emb0.T)
    emb1_2w = _to_rows(emb1.T)
    ips = _sc_ips(emb0_2w, emb1_2w, pidx_all, half_all)
    loss = _tc_loss(ips, mask)
    return loss[0, 0]


# speedup vs baseline: 2.8471x; 1.1595x over previous
import jax
import jax.numpy as jnp
from jax import lax
from jax.experimental import pallas as pl
from jax.experimental.pallas import tpu as pltpu
from jax.experimental.pallas import tpu_sc as plsc

_VOCAB = 1000000
_DIM = 64
_NEG = 5
_B = 16384

_NC = 2
_NS = 16
_NW = _NC * _NS
_ROWS_PER_W = _B // _NW
_CH = 128
_N_CHUNKS = _ROWS_PER_W // _CH
_NIDX = 2 + _NEG


def _sc_body(emb0_hbm, emb1_hbm, pidx_hbm, half_hbm, out_hbm,
             pidx_v, half_v, w_v, c_v, n_v, out_v, sem):
    wid = lax.axis_index("s") * _NC + lax.axis_index("c")
    base = wid * _ROWS_PER_W
    lane = lax.iota(jnp.int32, 16)

    def chunk_body(chunk, _):
        cbase = base + chunk * _CH
        pltpu.sync_copy(pidx_hbm.at[:, pl.ds(cbase, _CH)], pidx_v)
        pltpu.sync_copy(half_hbm.at[:, pl.ds(cbase, _CH)], half_v)
        cps = [
            pltpu.async_copy(emb0_hbm.at[pidx_v.at[0]], w_v, sem),
            pltpu.async_copy(emb1_hbm.at[pidx_v.at[1]], c_v, sem),
        ]
        for k in range(_NEG):
            cps.append(
                pltpu.async_copy(emb1_hbm.at[pidx_v.at[2 + k]], n_v.at[k],
                                 sem))
        for cp in cps:
            cp.wait()

        def group_body(g, _):
            rows = lane + g * 16
            off = [half_v[t, pl.ds(g * 16, 16)] * 64 for t in range(_NIDX)]
            accs = [jnp.zeros((16,), jnp.float32) for _ in range(6)]
            for d in range(_DIM):
                w = plsc.load_gather(w_v, [rows, off[0] + d])
                c = plsc.load_gather(c_v, [rows, off[1] + d])
                accs[0] = accs[0] + w * c
                for k in range(_NEG):
                    kk = jnp.full((16,), k, jnp.int32)
                    nk = plsc.load_gather(n_v, [kk, rows, off[2 + k] + d])
                    accs[1 + k] = accs[1 + k] + w * nk
            for t in range(6):
                out_v[t, pl.ds(g * 16, 16)] = accs[t]
            return 0

        lax.fori_loop(0, _CH // 16, group_body, 0)
        pltpu.sync_copy(out_v, out_hbm.at[:, pl.ds(cbase, _CH)])
        return 0

    lax.fori_loop(0, _N_CHUNKS, chunk_body, 0)


def _sc_ips(emb0_2w, emb1_2w, pidx_all, half_all):
    fn = pl.kernel(
        _sc_body,
        out_type=jax.ShapeDtypeStruct((6, _B), jnp.float32),
        mesh=plsc.VectorSubcoreMesh(core_axis_name="c", subcore_axis_name="s"),
        scratch_types=[
            pltpu.VMEM((_NIDX, _CH), jnp.int32),
            pltpu.VMEM((_NIDX, _CH), jnp.int32),
            pltpu.VMEM((_CH, 2 * _DIM), jnp.float32),
            pltpu.VMEM((_CH, 2 * _DIM), jnp.float32),
            pltpu.VMEM((_NEG, _CH, 2 * _DIM), jnp.float32),
            pltpu.VMEM((6, _CH), jnp.float32),
            pltpu.SemaphoreType.DMA,
        ],
        compiler_params=pltpu.CompilerParams(needs_layout_passes=False),
    )
    return fn(emb0_2w, emb1_2w, pidx_all, half_all)


_TRC = 2048
_NBLK = 245
_HALF = _TRC * _NBLK


def _tr_body(lo_ref, hi_ref, out_ref):
    cat = jnp.concatenate([lo_ref[...], hi_ref[...]], axis=0)
    eye = jnp.eye(2 * _DIM, dtype=jnp.float32)
    out_ref[...] = jax.lax.dot_general(
        cat, eye, (((0,), (0,)), ((), ())),
        preferred_element_type=jnp.float32)


def _to_rows(emb_t):
    hi_last = _VOCAB // _TRC
    return pl.pallas_call(
        _tr_body,
        grid=(_NBLK,),
        in_specs=[
            pl.BlockSpec((_DIM, _TRC), lambda i: (0, i)),
            pl.BlockSpec((_DIM, _TRC),
                         lambda i: (0, jnp.minimum(i + _NBLK, hi_last))),
        ],
        out_specs=pl.BlockSpec((_TRC, 128), lambda i: (i, 0)),
        out_shape=jax.ShapeDtypeStruct((_HALF, 128), jnp.float32),
    )(emb_t, emb_t)


def _loss_body(ips_ref, mask_ref, out_ref):
    ips = ips_ref[...]
    m = mask_ref[...]
    pos_y = jnp.clip(ips[0:1, :], -10.0, 10.0)
    neg_y = jnp.clip(-ips[1:6, :], -10.0, 10.0)
    pos_l = jnp.log1p(jnp.exp(-pos_y))
    neg_l = jnp.log1p(jnp.exp(-neg_y)) * m
    out_ref[0, 0] = jnp.sum(pos_l) + jnp.sum(neg_l)


def _tc_loss(ips, mask):
    return pl.pallas_call(
        _loss_body,
        out_shape=jax.ShapeDtypeStruct((1, 1), jnp.float32),
        out_specs=pl.BlockSpec(memory_space=pltpu.SMEM),
    )(ips, mask)


def kernel(data, emb0, emb1):
    cols = data.T
    idx_all = jnp.concatenate([cols[1:2], cols[0:1], cols[2:2 + _NEG]],
                              axis=0)
    half_all = (idx_all >= _HALF).astype(jnp.int32)
    pidx_all = idx_all - half_all * _HALF
    mask = cols[2 + _NEG:].astype(jnp.float32)
    emb0_2w = _to_rows(emb0.T)
    emb1_2w = _to_rows(emb1.T)
    ips = _sc_ips(emb0_2w, emb1_2w, pidx_all, half_all)
    loss = _tc_loss(ips, mask)
    return loss[0, 0]


# speedup vs baseline: 4.0347x; 1.4171x over previous
import jax
import jax.numpy as jnp
from jax import lax
from jax.experimental import pallas as pl
from jax.experimental.pallas import tpu as pltpu
from jax.experimental.pallas import tpu_sc as plsc

_VOCAB = 1000000
_DIM = 64
_NEG = 5
_B = 16384

_NC = 2
_NS = 16
_NW = _NC * _NS
_ROWS_PER_W = _B // _NW
_CH = 128
_N_CHUNKS = _ROWS_PER_W // _CH
_NIDX = 2 + _NEG


def _sc_body(emb0_hbm, emb1_hbm, pidx_hbm, half_hbm, out_hbm,
             pidx_v, half_v, w_v, c_v, n_v, out_v, sem):
    wid = lax.axis_index("s") * _NC + lax.axis_index("c")
    base = wid * _ROWS_PER_W
    lane = lax.iota(jnp.int32, 16)

    def chunk_body(chunk, _):
        cbase = base + chunk * _CH
        pltpu.sync_copy(pidx_hbm.at[:, pl.ds(cbase, _CH)], pidx_v)
        pltpu.sync_copy(half_hbm.at[:, pl.ds(cbase, _CH)], half_v)
        cps = [
            pltpu.async_copy(emb0_hbm.at[pidx_v.at[0]], w_v, sem),
            pltpu.async_copy(emb1_hbm.at[pidx_v.at[1]], c_v, sem),
        ]
        for k in range(_NEG):
            cps.append(
                pltpu.async_copy(emb1_hbm.at[pidx_v.at[2 + k]], n_v.at[k],
                                 sem))
        for cp in cps:
            cp.wait()

        def group_body(g, _):
            rows = lane + g * 16
            off = [half_v[t, pl.ds(g * 16, 16)] * 64 for t in range(_NIDX)]
            accs = [jnp.zeros((16,), jnp.float32) for _ in range(6)]
            for d in range(_DIM):
                w = plsc.load_gather(w_v, [rows, off[0] + d])
                c = plsc.load_gather(c_v, [rows, off[1] + d])
                accs[0] = accs[0] + w * c
                for k in range(_NEG):
                    kk = jnp.full((16,), k, jnp.int32)
                    nk = plsc.load_gather(n_v, [kk, rows, off[2 + k] + d])
                    accs[1 + k] = accs[1 + k] + w * nk
            for t in range(6):
                out_v[t, pl.ds(g * 16, 16)] = accs[t]
            return 0

        lax.fori_loop(0, _CH // 16, group_body, 0)
        pltpu.sync_copy(out_v, out_hbm.at[:, pl.ds(cbase, _CH)])
        return 0

    lax.fori_loop(0, _N_CHUNKS, chunk_body, 0)


def _sc_ips(emb0_2w, emb1_2w, pidx_all, half_all):
    fn = pl.kernel(
        _sc_body,
        out_type=jax.ShapeDtypeStruct((6, _B), jnp.float32),
        mesh=plsc.VectorSubcoreMesh(core_axis_name="c", subcore_axis_name="s"),
        scratch_types=[
            pltpu.VMEM((_NIDX, _CH), jnp.int32),
            pltpu.VMEM((_NIDX, _CH), jnp.int32),
            pltpu.VMEM((_CH, 2 * _DIM), jnp.float32),
            pltpu.VMEM((_CH, 2 * _DIM), jnp.float32),
            pltpu.VMEM((_NEG, _CH, 2 * _DIM), jnp.float32),
            pltpu.VMEM((6, _CH), jnp.float32),
            pltpu.SemaphoreType.DMA,
        ],
        compiler_params=pltpu.CompilerParams(needs_layout_passes=False),
    )
    return fn(emb0_2w, emb1_2w, pidx_all, half_all)


_TRC = 8192
_NBLK = 62
_HALF = _TRC * _NBLK


def _tr_body(lo_ref, hi_ref, out_ref):
    cat = jnp.concatenate([lo_ref[...], hi_ref[...]], axis=0)
    eye = jnp.eye(2 * _DIM, dtype=jnp.float32)
    out_ref[...] = jax.lax.dot_general(
        cat, eye, (((0,), (0,)), ((), ())),
        preferred_element_type=jnp.float32)


def _to_rows(emb_t):
    hi_last = _VOCAB // _TRC
    return pl.pallas_call(
        _tr_body,
        grid=(_NBLK,),
        in_specs=[
            pl.BlockSpec((_DIM, _TRC), lambda i: (0, i)),
            pl.BlockSpec((_DIM, _TRC),
                         lambda i: (0, jnp.minimum(i + _NBLK, hi_last))),
        ],
        out_specs=pl.BlockSpec((_TRC, 128), lambda i: (i, 0)),
        out_shape=jax.ShapeDtypeStruct((_HALF, 128), jnp.float32),
    )(emb_t, emb_t)


def _loss_body(ips_ref, mask_ref, out_ref):
    ips = ips_ref[...]
    m = mask_ref[...]
    pos_y = jnp.clip(ips[0:1, :], -10.0, 10.0)
    neg_y = jnp.clip(-ips[1:6, :], -10.0, 10.0)
    pos_l = jnp.log1p(jnp.exp(-pos_y))
    neg_l = jnp.log1p(jnp.exp(-neg_y)) * m
    out_ref[0, 0] = jnp.sum(pos_l) + jnp.sum(neg_l)


def _tc_loss(ips, mask):
    return pl.pallas_call(
        _loss_body,
        out_shape=jax.ShapeDtypeStruct((1, 1), jnp.float32),
        out_specs=pl.BlockSpec(memory_space=pltpu.SMEM),
    )(ips, mask)


def kernel(data, emb0, emb1):
    cols = data.T
    idx_all = jnp.concatenate([cols[1:2], cols[0:1], cols[2:2 + _NEG]],
                              axis=0)
    half_all = (idx_all >= _HALF).astype(jnp.int32)
    pidx_all = idx_all - half_all * _HALF
    mask = cols[2 + _NEG:].astype(jnp.float32)
    emb0_2w = _to_rows(emb0.T)
    emb1_2w = _to_rows(emb1.T)
    ips = _sc_ips(emb0_2w, emb1_2w, pidx_all, half_all)
    loss = _tc_loss(ips, mask)
    return loss[0, 0]


# speedup vs baseline: 4.1198x; 1.0211x over previous
import jax
import jax.numpy as jnp
from jax import lax
from jax.experimental import pallas as pl
from jax.experimental.pallas import tpu as pltpu
from jax.experimental.pallas import tpu_sc as plsc

_VOCAB = 1000000
_DIM = 64
_NEG = 5
_B = 16384

_NC = 2
_NS = 16
_NW = _NC * _NS
_ROWS_PER_W = _B // _NW
_CH = 128
_N_CHUNKS = _ROWS_PER_W // _CH
_NIDX = 2 + _NEG


def _sc_body(emb0_hbm, emb1_hbm, pidx_hbm, half_hbm, out_hbm,
             pidx_v, half_v, w_v, c_v, n_v, out_v, sem):
    wid = lax.axis_index("s") * _NC + lax.axis_index("c")
    base = wid * _ROWS_PER_W
    lane = lax.iota(jnp.int32, 16)

    def chunk_body(chunk, _):
        cbase = base + chunk * _CH
        pltpu.sync_copy(pidx_hbm.at[:, pl.ds(cbase, _CH)], pidx_v)
        pltpu.sync_copy(half_hbm.at[:, pl.ds(cbase, _CH)], half_v)
        cps = [
            pltpu.async_copy(emb0_hbm.at[pidx_v.at[0]], w_v, sem),
            pltpu.async_copy(emb1_hbm.at[pidx_v.at[1]], c_v, sem),
        ]
        for k in range(_NEG):
            cps.append(
                pltpu.async_copy(emb1_hbm.at[pidx_v.at[2 + k]], n_v.at[k],
                                 sem))
        for cp in cps:
            cp.wait()

        def group_body(g, _):
            rows = lane + g * 16
            off = [half_v[t, pl.ds(g * 16, 16)] * 64 for t in range(_NIDX)]
            accs = [jnp.zeros((16,), jnp.float32) for _ in range(6)]
            for d in range(_DIM):
                w = plsc.load_gather(w_v, [rows, off[0] + d])
                c = plsc.load_gather(c_v, [rows, off[1] + d])
                accs[0] = accs[0] + w * c
                for k in range(_NEG):
                    kk = jnp.full((16,), k, jnp.int32)
                    nk = plsc.load_gather(n_v, [kk, rows, off[2 + k] + d])
                    accs[1 + k] = accs[1 + k] + w * nk
            for t in range(6):
                out_v[t, pl.ds(g * 16, 16)] = accs[t]
            return 0

        lax.fori_loop(0, _CH // 16, group_body, 0)
        pltpu.sync_copy(out_v, out_hbm.at[:, pl.ds(cbase, _CH)])
        return 0

    lax.fori_loop(0, _N_CHUNKS, chunk_body, 0)


def _sc_ips(emb0_2w, emb1_2w, pidx_all, half_all):
    fn = pl.kernel(
        _sc_body,
        out_type=jax.ShapeDtypeStruct((6, _B), jnp.float32),
        mesh=plsc.VectorSubcoreMesh(core_axis_name="c", subcore_axis_name="s"),
        scratch_types=[
            pltpu.VMEM((_NIDX, _CH), jnp.int32),
            pltpu.VMEM((_NIDX, _CH), jnp.int32),
            pltpu.VMEM((_CH, 2 * _DIM), jnp.float32),
            pltpu.VMEM((_CH, 2 * _DIM), jnp.float32),
            pltpu.VMEM((_NEG, _CH, 2 * _DIM), jnp.float32),
            pltpu.VMEM((6, _CH), jnp.float32),
            pltpu.SemaphoreType.DMA,
        ],
        compiler_params=pltpu.CompilerParams(needs_layout_passes=False),
    )
    return fn(emb0_2w, emb1_2w, pidx_all, half_all)


_TRC = 16384
_NBLK = 31
_HALF = _TRC * _NBLK


def _tr_body(lo_ref, hi_ref, out_ref):
    cat = jnp.concatenate([lo_ref[...], hi_ref[...]], axis=0)
    eye = jnp.eye(2 * _DIM, dtype=jnp.float32)
    out_ref[...] = jax.lax.dot_general(
        cat, eye, (((0,), (0,)), ((), ())),
        preferred_element_type=jnp.float32)


def _to_rows(emb_t):
    hi_last = _VOCAB // _TRC
    return pl.pallas_call(
        _tr_body,
        grid=(_NBLK,),
        in_specs=[
            pl.BlockSpec((_DIM, _TRC), lambda i: (0, i)),
            pl.BlockSpec((_DIM, _TRC),
                         lambda i: (0, jnp.minimum(i + _NBLK, hi_last))),
        ],
        out_specs=pl.BlockSpec((_TRC, 128), lambda i: (i, 0)),
        out_shape=jax.ShapeDtypeStruct((_HALF, 128), jnp.float32),
    )(emb_t, emb_t)


def _loss_body(ips_ref, mask_ref, out_ref):
    ips = ips_ref[...]
    m = mask_ref[...]
    pos_y = jnp.clip(ips[0:1, :], -10.0, 10.0)
    neg_y = jnp.clip(-ips[1:6, :], -10.0, 10.0)
    pos_l = jnp.log1p(jnp.exp(-pos_y))
    neg_l = jnp.log1p(jnp.exp(-neg_y)) * m
    out_ref[0, 0] = jnp.sum(pos_l) + jnp.sum(neg_l)


def _tc_loss(ips, mask):
    return pl.pallas_call(
        _loss_body,
        out_shape=jax.ShapeDtypeStruct((1, 1), jnp.float32),
        out_specs=pl.BlockSpec(memory_space=pltpu.SMEM),
    )(ips, mask)


def kernel(data, emb0, emb1):
    cols = data.T
    idx_all = jnp.concatenate([cols[1:2], cols[0:1], cols[2:2 + _NEG]],
                              axis=0)
    half_all = (idx_all >= _HALF).astype(jnp.int32)
    pidx_all = idx_all - half_all * _HALF
    mask = cols[2 + _NEG:].astype(jnp.float32)
    emb0_2w = _to_rows(emb0.T)
    emb1_2w = _to_rows(emb1.T)
    ips = _sc_ips(emb0_2w, emb1_2w, pidx_all, half_all)
    loss = _tc_loss(ips, mask)
    return loss[0, 0]


# speedup vs baseline: 4.2789x; 1.0386x over previous
import jax
import jax.numpy as jnp
from jax import lax
from jax.experimental import pallas as pl
from jax.experimental.pallas import tpu as pltpu
from jax.experimental.pallas import tpu_sc as plsc

_VOCAB = 1000000
_DIM = 64
_NEG = 5
_B = 16384

_NC = 2
_NS = 16
_NW = _NC * _NS
_ROWS_PER_W = _B // _NW
_CH = 64
_N_CHUNKS = _ROWS_PER_W // _CH
_NIDX = 2 + _NEG


def _sc_body(emb0_hbm, emb1_hbm, pidx_hbm, half_hbm, out_hbm,
             pidx_v, half_v, rows_v, out_v, sem0, sem1):
    wid = lax.axis_index("s") * _NC + lax.axis_index("c")
    base = wid * _ROWS_PER_W
    lane = lax.iota(jnp.int32, 16)
    sems = (sem0, sem1)

    pltpu.sync_copy(pidx_hbm.at[:, pl.ds(base, _ROWS_PER_W)], pidx_v)
    pltpu.sync_copy(half_hbm.at[:, pl.ds(base, _ROWS_PER_W)], half_v)

    def fire(k, b, sem):
        pltpu.async_copy(
            emb0_hbm.at[pidx_v.at[0, pl.ds(k * _CH, _CH)]],
            rows_v.at[b, 0], sem)
        for t in range(1, _NIDX):
            pltpu.async_copy(
                emb1_hbm.at[pidx_v.at[t, pl.ds(k * _CH, _CH)]],
                rows_v.at[b, t], sem)

    def drain(b, sem):
        for t in range(_NIDX):
            pltpu.make_async_copy(
                emb0_hbm.at[pl.ds(0, _CH)], rows_v.at[b, t], sem).wait()

    def compute(k, b):
        bb = jnp.full((16,), b, jnp.int32)

        def group_body(g, _):
            r0 = k * _CH + g * 16
            rows = lane + g * 16
            off = [half_v[t, pl.ds(r0, 16)] * _DIM for t in range(_NIDX)]
            accs = [jnp.zeros((16,), jnp.float32) for _ in range(6)]
            for d in range(_DIM):
                w = plsc.load_gather(
                    rows_v, [bb, jnp.full((16,), 0, jnp.int32), rows,
                             off[0] + d])
                c = plsc.load_gather(
                    rows_v, [bb, jnp.full((16,), 1, jnp.int32), rows,
                             off[1] + d])
                accs[0] = accs[0] + w * c
                for t in range(_NEG):
                    nk = plsc.load_gather(
                        rows_v, [bb, jnp.full((16,), 2 + t, jnp.int32), rows,
                                 off[2 + t] + d])
                    accs[1 + t] = accs[1 + t] + w * nk
            for t in range(6):
                out_v[t, pl.ds(r0, 16)] = accs[t]
            return 0

        lax.fori_loop(0, _CH // 16, group_body, 0)

    fire(0, 0, sems[0])

    def pair_body(c2, _):
        for b in range(2):
            k = c2 * 2 + b

            @pl.when(k + 1 < _N_CHUNKS)
            def _():
                fire(k + 1, 1 - b, sems[1 - b])

            drain(b, sems[b])
            compute(k, b)
        return 0

    lax.fori_loop(0, _N_CHUNKS // 2, pair_body, 0)
    pltpu.sync_copy(out_v, out_hbm.at[:, pl.ds(base, _ROWS_PER_W)])


def _sc_ips(emb0_2w, emb1_2w, pidx_all, half_all):
    fn = pl.kernel(
        _sc_body,
        out_type=jax.ShapeDtypeStruct((6, _B), jnp.float32),
        mesh=plsc.VectorSubcoreMesh(core_axis_name="c", subcore_axis_name="s"),
        scratch_types=[
            pltpu.VMEM((_NIDX, _ROWS_PER_W), jnp.int32),
            pltpu.VMEM((_NIDX, _ROWS_PER_W), jnp.int32),
            pltpu.VMEM((2, _NIDX, _CH, 2 * _DIM), jnp.float32),
            pltpu.VMEM((6, _ROWS_PER_W), jnp.float32),
            pltpu.SemaphoreType.DMA,
            pltpu.SemaphoreType.DMA,
        ],
        compiler_params=pltpu.CompilerParams(needs_layout_passes=False),
    )
    return fn(emb0_2w, emb1_2w, pidx_all, half_all)


_TRC = 16384
_NBLK = 31
_HALF = _TRC * _NBLK


def _tr_body(lo_ref, hi_ref, out_ref):
    cat = jnp.concatenate([lo_ref[...], hi_ref[...]], axis=0)
    eye = jnp.eye(2 * _DIM, dtype=jnp.float32)
    out_ref[...] = jax.lax.dot_general(
        cat, eye, (((0,), (0,)), ((), ())),
        preferred_element_type=jnp.float32)


def _to_rows(emb_t):
    hi_last = _VOCAB // _TRC
    return pl.pallas_call(
        _tr_body,
        grid=(_NBLK,),
        in_specs=[
            pl.BlockSpec((_DIM, _TRC), lambda i: (0, i)),
            pl.BlockSpec((_DIM, _TRC),
                         lambda i: (0, jnp.minimum(i + _NBLK, hi_last))),
        ],
        out_specs=pl.BlockSpec((_TRC, 128), lambda i: (i, 0)),
        out_shape=jax.ShapeDtypeStruct((_HALF, 128), jnp.float32),
    )(emb_t, emb_t)


def _loss_body(ips_ref, mask_ref, out_ref):
    ips = ips_ref[...]
    m = mask_ref[...]
    pos_y = jnp.clip(ips[0:1, :], -10.0, 10.0)
    neg_y = jnp.clip(-ips[1:6, :], -10.0, 10.0)
    pos_l = jnp.log1p(jnp.exp(-pos_y))
    neg_l = jnp.log1p(jnp.exp(-neg_y)) * m
    out_ref[0, 0] = jnp.sum(pos_l) + jnp.sum(neg_l)


def _tc_loss(ips, mask):
    return pl.pallas_call(
        _loss_body,
        out_shape=jax.ShapeDtypeStruct((1, 1), jnp.float32),
        out_specs=pl.BlockSpec(memory_space=pltpu.SMEM),
    )(ips, mask)


def kernel(data, emb0, emb1):
    cols = data.T
    idx_all = jnp.concatenate([cols[1:2], cols[0:1], cols[2:2 + _NEG]],
                              axis=0)
    half_all = (idx_all >= _HALF).astype(jnp.int32)
    pidx_all = idx_all - half_all * _HALF
    mask = cols[2 + _NEG:].astype(jnp.float32)
    emb0_2w = _to_rows(emb0.T)
    emb1_2w = _to_rows(emb1.T)
    ips = _sc_ips(emb0_2w, emb1_2w, pidx_all, half_all)
    loss = _tc_loss(ips, mask)
    return loss[0, 0]


# speedup vs baseline: 4.2906x; 1.0027x over previous
import jax
import jax.numpy as jnp
from jax import lax
from jax.experimental import pallas as pl
from jax.experimental.pallas import tpu as pltpu
from jax.experimental.pallas import tpu_sc as plsc

_VOCAB = 1000000
_DIM = 64
_NEG = 5
_B = 16384

_NC = 2
_NS = 16
_NW = _NC * _NS
_ROWS_PER_W = _B // _NW
_CH = 32
_N_CHUNKS = _ROWS_PER_W // _CH
_NIDX = 2 + _NEG


def _sc_body(emb0_hbm, emb1_hbm, pidx_hbm, half_hbm, out_hbm,
             pidx_v, half_v, rows_v, out_v, sem0, sem1):
    wid = lax.axis_index("s") * _NC + lax.axis_index("c")
    base = wid * _ROWS_PER_W
    lane = lax.iota(jnp.int32, 16)
    sems = (sem0, sem1)

    pltpu.sync_copy(pidx_hbm.at[:, pl.ds(base, _ROWS_PER_W)], pidx_v)
    pltpu.sync_copy(half_hbm.at[:, pl.ds(base, _ROWS_PER_W)], half_v)

    def fire(k, b, sem):
        pltpu.async_copy(
            emb0_hbm.at[pidx_v.at[0, pl.ds(k * _CH, _CH)]],
            rows_v.at[b, 0], sem)
        for t in range(1, _NIDX):
            pltpu.async_copy(
                emb1_hbm.at[pidx_v.at[t, pl.ds(k * _CH, _CH)]],
                rows_v.at[b, t], sem)

    def drain(b, sem):
        for t in range(_NIDX):
            pltpu.make_async_copy(
                emb0_hbm.at[pl.ds(0, _CH)], rows_v.at[b, t], sem).wait()

    def compute(k, b):
        bb = jnp.full((16,), b, jnp.int32)

        @plsc.parallel_loop(0, _CH // 16, unroll=2)
        def group_body(g):
            r0 = k * _CH + g * 16
            rows = lane + g * 16
            off = [half_v[t, pl.ds(r0, 16)] * _DIM for t in range(_NIDX)]
            accs = [jnp.zeros((16,), jnp.float32) for _ in range(6)]
            for d in range(_DIM):
                w = plsc.load_gather(
                    rows_v, [bb, jnp.full((16,), 0, jnp.int32), rows,
                             off[0] + d])
                c = plsc.load_gather(
                    rows_v, [bb, jnp.full((16,), 1, jnp.int32), rows,
                             off[1] + d])
                accs[0] = accs[0] + w * c
                for t in range(_NEG):
                    nk = plsc.load_gather(
                        rows_v, [bb, jnp.full((16,), 2 + t, jnp.int32), rows,
                                 off[2 + t] + d])
                    accs[1 + t] = accs[1 + t] + w * nk
            for t in range(6):
                out_v[t, pl.ds(r0, 16)] = accs[t]

    fire(0, 0, sems[0])

    def pair_body(c2, _):
        for b in range(2):
            k = c2 * 2 + b

            @pl.when(k + 1 < _N_CHUNKS)
            def _():
                fire(k + 1, 1 - b, sems[1 - b])

            drain(b, sems[b])
            compute(k, b)
        return 0

    lax.fori_loop(0, _N_CHUNKS // 2, pair_body, 0)
    pltpu.sync_copy(out_v, out_hbm.at[:, pl.ds(base, _ROWS_PER_W)])


def _sc_ips(emb0_2w, emb1_2w, pidx_all, half_all):
    fn = pl.kernel(
        _sc_body,
        out_type=jax.ShapeDtypeStruct((6, _B), jnp.float32),
        mesh=plsc.VectorSubcoreMesh(core_axis_name="c", subcore_axis_name="s"),
        scratch_types=[
            pltpu.VMEM((_NIDX, _ROWS_PER_W), jnp.int32),
            pltpu.VMEM((_NIDX, _ROWS_PER_W), jnp.int32),
            pltpu.VMEM((2, _NIDX, _CH, 2 * _DIM), jnp.float32),
            pltpu.VMEM((6, _ROWS_PER_W), jnp.float32),
            pltpu.SemaphoreType.DMA,
            pltpu.SemaphoreType.DMA,
        ],
        compiler_params=pltpu.CompilerParams(needs_layout_passes=False),
    )
    return fn(emb0_2w, emb1_2w, pidx_all, half_all)


_TRC = 16384
_NBLK = 31
_HALF = _TRC * _NBLK


def _tr_body(lo_ref, hi_ref, out_ref):
    cat = jnp.concatenate([lo_ref[...], hi_ref[...]], axis=0)
    eye = jnp.eye(2 * _DIM, dtype=jnp.float32)
    out_ref[...] = jax.lax.dot_general(
        cat, eye, (((0,), (0,)), ((), ())),
        preferred_element_type=jnp.float32)


def _to_rows(emb_t):
    hi_last = _VOCAB // _TRC
    return pl.pallas_call(
        _tr_body,
        grid=(_NBLK,),
        in_specs=[
            pl.BlockSpec((_DIM, _TRC), lambda i: (0, i)),
            pl.BlockSpec((_DIM, _TRC),
                         lambda i: (0, jnp.minimum(i + _NBLK, hi_last))),
        ],
        out_specs=pl.BlockSpec((_TRC, 128), lambda i: (i, 0)),
        out_shape=jax.ShapeDtypeStruct((_HALF, 128), jnp.float32),
    )(emb_t, emb_t)


def _loss_body(ips_ref, mask_ref, out_ref):
    ips = ips_ref[...]
    m = mask_ref[...]
    pos_y = jnp.clip(ips[0:1, :], -10.0, 10.0)
    neg_y = jnp.clip(-ips[1:6, :], -10.0, 10.0)
    pos_l = jnp.log1p(jnp.exp(-pos_y))
    neg_l = jnp.log1p(jnp.exp(-neg_y)) * m
    out_ref[0, 0] = jnp.sum(pos_l) + jnp.sum(neg_l)


def _tc_loss(ips, mask):
    return pl.pallas_call(
        _loss_body,
        out_shape=jax.ShapeDtypeStruct((1, 1), jnp.float32),
        out_specs=pl.BlockSpec(memory_space=pltpu.SMEM),
    )(ips, mask)


def kernel(data, emb0, emb1):
    cols = data.T
    idx_all = jnp.concatenate([cols[1:2], cols[0:1], cols[2:2 + _NEG]],
                              axis=0)
    half_all = (idx_all >= _HALF).astype(jnp.int32)
    pidx_all = idx_all - half_all * _HALF
    mask = cols[2 + _NEG:].astype(jnp.float32)
    emb0_2w = _to_rows(emb0.T)
    emb1_2w = _to_rows(emb1.T)
    ips = _sc_ips(emb0_2w, emb1_2w, pidx_all, half_all)
    loss = _tc_loss(ips, mask)
    return loss[0, 0]


# speedup vs baseline: 4.2909x; 1.0001x over previous
import jax
import jax.numpy as jnp
from jax import lax
from jax.experimental import pallas as pl
from jax.experimental.pallas import tpu as pltpu
from jax.experimental.pallas import tpu_sc as plsc

_VOCAB = 1000000
_DIM = 64
_NEG = 5
_B = 16384

_NC = 2
_NS = 16
_NW = _NC * _NS
_ROWS_PER_W = _B // _NW
_CH = 32
_N_CHUNKS = _ROWS_PER_W // _CH
_NIDX = 2 + _NEG


def _sc_body(emb0_hbm, emb1_hbm, pidx_hbm, half_hbm, out_hbm,
             pidx_v, half_v, rows_v, out_v, sem0, sem1):
    wid = lax.axis_index("s") * _NC + lax.axis_index("c")
    base = wid * _ROWS_PER_W
    lane = lax.iota(jnp.int32, 16)
    sems = (sem0, sem1)

    pltpu.sync_copy(pidx_hbm.at[:, pl.ds(base, _ROWS_PER_W)], pidx_v)
    pltpu.sync_copy(half_hbm.at[:, pl.ds(base, _ROWS_PER_W)], half_v)

    def fire(k, b, sem):
        pltpu.async_copy(
            emb0_hbm.at[pidx_v.at[0, pl.ds(k * _CH, _CH)]],
            rows_v.at[b, 0], sem)
        for t in range(1, _NIDX):
            pltpu.async_copy(
                emb1_hbm.at[pidx_v.at[t, pl.ds(k * _CH, _CH)]],
                rows_v.at[b, t], sem)

    def drain(b, sem):
        for t in range(_NIDX):
            pltpu.make_async_copy(
                emb0_hbm.at[pl.ds(0, _CH)], rows_v.at[b, t], sem).wait()

    def compute(k, b):
        bb = jnp.full((16,), b, jnp.int32)

        @plsc.parallel_loop(0, _CH // 16, unroll=2)
        def group_body(g):
            r0 = k * _CH + g * 16
            rows = lane + g * 16
            off = [half_v[t, pl.ds(r0, 16)] * _DIM for t in range(_NIDX)]
            accs = [jnp.zeros((16,), jnp.float32) for _ in range(6)]
            for d in range(_DIM):
                w = plsc.load_gather(
                    rows_v, [bb, jnp.full((16,), 0, jnp.int32), rows,
                             off[0] + d])
                c = plsc.load_gather(
                    rows_v, [bb, jnp.full((16,), 1, jnp.int32), rows,
                             off[1] + d])
                accs[0] = accs[0] + w * c
                for t in range(_NEG):
                    nk = plsc.load_gather(
                        rows_v, [bb, jnp.full((16,), 2 + t, jnp.int32), rows,
                                 off[2 + t] + d])
                    accs[1 + t] = accs[1 + t] + w * nk
            for t in range(6):
                out_v[t, pl.ds(r0, 16)] = accs[t]

    fire(0, 0, sems[0])

    def pair_body(c2, _):
        for b in range(2):
            k = c2 * 2 + b

            @pl.when(k + 1 < _N_CHUNKS)
            def _():
                fire(k + 1, 1 - b, sems[1 - b])

            drain(b, sems[b])
            compute(k, b)
        return 0

    lax.fori_loop(0, _N_CHUNKS // 2, pair_body, 0)
    pltpu.sync_copy(out_v, out_hbm.at[:, pl.ds(base, _ROWS_PER_W)])


def _sc_ips(emb0_2w, emb1_2w, pidx_all, half_all):
    fn = pl.kernel(
        _sc_body,
        out_type=jax.ShapeDtypeStruct((6, _B), jnp.float32),
        mesh=plsc.VectorSubcoreMesh(core_axis_name="c", subcore_axis_name="s"),
        scratch_types=[
            pltpu.VMEM((_NIDX, _ROWS_PER_W), jnp.int32),
            pltpu.VMEM((_NIDX, _ROWS_PER_W), jnp.int32),
            pltpu.VMEM((2, _NIDX, _CH, 2 * _DIM), jnp.float32),
            pltpu.VMEM((6, _ROWS_PER_W), jnp.float32),
            pltpu.SemaphoreType.DMA,
            pltpu.SemaphoreType.DMA,
        ],
        compiler_params=pltpu.CompilerParams(needs_layout_passes=False),
    )
    return fn(emb0_2w, emb1_2w, pidx_all, half_all)


_TRC = 24576
_NBLK = 21
_HALF = _TRC * _NBLK


def _tr_body(lo_ref, hi_ref, out_ref):
    cat = jnp.concatenate([lo_ref[...], hi_ref[...]], axis=0)
    eye = jnp.eye(2 * _DIM, dtype=jnp.float32)
    out_ref[...] = jax.lax.dot_general(
        cat, eye, (((0,), (0,)), ((), ())),
        preferred_element_type=jnp.float32)


def _to_rows(emb_t):
    hi_last = _VOCAB // _TRC
    return pl.pallas_call(
        _tr_body,
        grid=(_NBLK,),
        in_specs=[
            pl.BlockSpec((_DIM, _TRC), lambda i: (0, i)),
            pl.BlockSpec((_DIM, _TRC),
                         lambda i: (0, jnp.minimum(i + _NBLK, hi_last))),
        ],
        out_specs=pl.BlockSpec((_TRC, 128), lambda i: (i, 0)),
        out_shape=jax.ShapeDtypeStruct((_HALF, 128), jnp.float32),
    )(emb_t, emb_t)


def _loss_body(ips_ref, mask_ref, out_ref):
    ips = ips_ref[...]
    m = mask_ref[...]
    pos_y = jnp.clip(ips[0:1, :], -10.0, 10.0)
    neg_y = jnp.clip(-ips[1:6, :], -10.0, 10.0)
    pos_l = jnp.log1p(jnp.exp(-pos_y))
    neg_l = jnp.log1p(jnp.exp(-neg_y)) * m
    out_ref[0, 0] = jnp.sum(pos_l) + jnp.sum(neg_l)


def _tc_loss(ips, mask):
    return pl.pallas_call(
        _loss_body,
        out_shape=jax.ShapeDtypeStruct((1, 1), jnp.float32),
        out_specs=pl.BlockSpec(memory_space=pltpu.SMEM),
    )(ips, mask)


def kernel(data, emb0, emb1):
    cols = data.T
    idx_all = jnp.concatenate([cols[1:2], cols[0:1], cols[2:2 + _NEG]],
                              axis=0)
    half_all = (idx_all >= _HALF).astype(jnp.int32)
    pidx_all = idx_all - half_all * _HALF
    mask = cols[2 + _NEG:].astype(jnp.float32)
    emb0_2w = _to_rows(emb0.T)
    emb1_2w = _to_rows(emb1.T)
    ips = _sc_ips(emb0_2w, emb1_2w, pidx_all, half_all)
    loss = _tc_loss(ips, mask)
    return loss[0, 0]
